# baseline (device time: 25490 ns/iter reference)
import jax
import jax.numpy as jnp
from jax import lax
from jax.experimental import pallas as pl
from jax.experimental.pallas import tpu as pltpu


def kernel(Q, K, V):
    b, q, h, d = Q.shape
    _, kv, _, _ = K.shape
    hd = h * d
    scale = d ** -0.5

    Q2 = Q.reshape(b, hd, 1)
    K2 = K.reshape(b, kv, hd)
    V2 = V.reshape(b, kv, hd)

    def body(q_ref, k_ref, v_ref, o_ref,
             u_send, u_recv, ml_send, ml_recv, send_sems, recv_sems):
        my_x = lax.axis_index("x")
        my_y = lax.axis_index("y")
        nbr = (1 - my_x, my_y)
        bi = pl.program_id(0)

        rows = lax.broadcasted_iota(jnp.int32, (hd, h), 0) // d
        cols = lax.broadcasted_iota(jnp.int32, (hd, h), 1)
        sel = (rows == cols)

        barrier = pltpu.get_barrier_semaphore()

        @pl.when(bi == 0)
        def _():
            pl.semaphore_signal(barrier, inc=1, device_id=nbr,
                                device_id_type=pl.DeviceIdType.MESH)

        Qblk = jnp.where(sel, q_ref[bi], 0.0)
        Sb = jnp.dot(k_ref[0], Qblk,
                     preferred_element_type=jnp.float32) * scale
        mb = jnp.max(Sb, axis=0, keepdims=True)
        Pb = jnp.exp(Sb - mb)
        lb = jnp.sum(Pb, axis=0, keepdims=True)
        R = lax.dot_general(Pb, v_ref[0],
                            dimension_numbers=(((0,), (0,)), ((), ())),
                            preferred_element_type=jnp.float32)
        Ub = jnp.sum(R * jnp.where(sel, 1.0, 0.0).T, axis=0,
                     keepdims=True)
        u_send[pl.ds(bi, 1), :] = Ub
        ml_send[0, pl.ds(bi, 1), :] = mb
        ml_send[1, pl.ds(bi, 1), :] = lb

        @pl.when(bi == b - 1)
        def _():
            pl.semaphore_wait(barrier, 1)
            rdma_u = pltpu.make_async_remote_copy(
                src_ref=u_send, dst_ref=u_recv,
                send_sem=send_sems.at[0], recv_sem=recv_sems.at[0],
                device_id=nbr, device_id_type=pl.DeviceIdType.MESH,
            )
            rdma_ml = pltpu.make_async_remote_copy(
                src_ref=ml_send, dst_ref=ml_recv,
                send_sem=send_sems.at[1], recv_sem=recv_sems.at[1],
                device_id=nbr, device_id_type=pl.DeviceIdType.MESH,
            )
            rdma_u.start()
            rdma_ml.start()
            rdma_u.wait()
            rdma_ml.wait()

            m_loc = ml_send[0]
            l_loc = ml_send[1]
            m_rem = ml_recv[0]
            l_rem = ml_recv[1]
            m_new = jnp.maximum(m_loc, m_rem)
            a = jnp.exp(m_loc - m_new)
            c = jnp.exp(m_rem - m_new)
            l_new = l_loc * a + l_rem * c
            onehot_t = jnp.where(sel, 1.0, 0.0).T
            a_exp = jnp.dot(a, onehot_t, preferred_element_type=jnp.float32)
            c_exp = jnp.dot(c, onehot_t, preferred_element_type=jnp.float32)
            l_exp = jnp.dot(l_new, onehot_t,
                            preferred_element_type=jnp.float32)
            o_ref[:, :] = (u_send[:, :] * a_exp + u_recv[:, :] * c_exp) / l_exp

    out = pl.pallas_call(
        body,
        grid=(b,),
        out_shape=jax.ShapeDtypeStruct((b, hd), jnp.float32),
        in_specs=[
            pl.BlockSpec(memory_space=pltpu.VMEM),
            pl.BlockSpec((1, kv, hd), lambda i: (i, 0, 0)),
            pl.BlockSpec((1, kv, hd), lambda i: (i, 0, 0)),
        ],
        out_specs=pl.BlockSpec((b, hd), lambda i: (0, 0)),
        scratch_shapes=[
            pltpu.VMEM((b, hd), jnp.float32),
            pltpu.VMEM((b, hd), jnp.float32),
            pltpu.VMEM((2, b, h), jnp.float32),
            pltpu.VMEM((2, b, h), jnp.float32),
            pltpu.SemaphoreType.DMA((2,)),
            pltpu.SemaphoreType.DMA((2,)),
        ],
        compiler_params=pltpu.CompilerParams(collective_id=0),
    )(Q2, K2, V2)
    return out.reshape(b, q, h, d)


# device time: 17644 ns/iter; 1.4447x vs baseline; 1.4447x over previous
import jax
import jax.numpy as jnp
from jax import lax
from jax.experimental import pallas as pl
from jax.experimental.pallas import tpu as pltpu


def kernel(Q, K, V):
    b, q, h, d = Q.shape
    _, kv, _, _ = K.shape
    hd = h * d
    b2 = b // 2
    scale = d ** -0.5
    rows = hd + 2 * h

    KT = K.transpose(0, 2, 3, 1).reshape(b, hd, kv)
    VT = V.transpose(0, 2, 3, 1).reshape(b, hd, kv)

    ysel = lax.axis_index("y")
    KTh = lax.dynamic_slice_in_dim(KT, ysel * b2, b2, axis=0).astype(
        jnp.bfloat16)
    VTh = lax.dynamic_slice_in_dim(VT, ysel * b2, b2, axis=0).astype(
        jnp.bfloat16)

    def body(q_ref, k_ref, v_ref, o_ref,
             pack_send, pack_recv, send_sems, recv_sems):
        my_x = lax.axis_index("x")
        my_y = lax.axis_index("y")
        peers = [(1 - my_x, my_y), (my_x, 1 - my_y), (1 - my_x, 1 - my_y)]

        barrier = pltpu.get_barrier_semaphore()
        for p in peers:
            pl.semaphore_signal(barrier, inc=1, device_id=p,
                                device_id_type=pl.DeviceIdType.MESH)

        bigmask = (lax.broadcasted_iota(jnp.int32, (h, hd), 1) // d
                   == lax.broadcasted_iota(jnp.int32, (h, hd), 0)
                   ).astype(jnp.float32)
        onehot = (lax.broadcasted_iota(jnp.int32, (hd, h), 0) // d
                  == lax.broadcasted_iota(jnp.int32, (hd, h), 1)
                  ).astype(jnp.float32)
        onehot_bf = onehot.astype(jnp.bfloat16)
        ones_kv = jnp.ones((kv, 1), dtype=jnp.bfloat16)

        pbs = []
        for i in range(b2):
            bidx = my_y * b2 + i
            Qb = q_ref[bidx, 0]
            Qrep = jnp.concatenate([Qb] * h, axis=1)
            Qblk_t = (Qrep * bigmask).astype(jnp.bfloat16)
            Sb = jnp.dot(Qblk_t, k_ref[i],
                         preferred_element_type=jnp.float32) * scale
            mb = jnp.max(Sb, axis=1, keepdims=True)
            Pb = jnp.exp(Sb - mb)
            lb = jnp.sum(Pb, axis=1, keepdims=True)
            pbs.append(Pb.astype(jnp.bfloat16))
            pack_send[hd:hd + h, i:i + 1] = mb
            pack_send[hd + h:rows, i:i + 1] = lb

        pcat = jnp.concatenate(pbs, axis=1)
        pexp = jnp.dot(onehot_bf, pcat,
                       preferred_element_type=jnp.float32
                       ).astype(jnp.bfloat16)
        vcat = jnp.concatenate([v_ref[i] for i in range(b2)],
                               axis=1)
        blkones = (lax.broadcasted_iota(jnp.int32, (b2 * kv, b2), 0) // kv
                   == lax.broadcasted_iota(jnp.int32, (b2 * kv, b2), 1)
                   ).astype(jnp.bfloat16)
        u_all = jnp.dot(pexp * vcat, blkones,
                        preferred_element_type=jnp.float32)
        pack_send[0:hd, :] = u_all

        pl.semaphore_wait(barrier, 3)

        rdmas = []
        for s, p in enumerate(peers):
            r = pltpu.make_async_remote_copy(
                src_ref=pack_send, dst_ref=pack_recv.at[s],
                send_sem=send_sems.at[s], recv_sem=recv_sems.at[s],
                device_id=p, device_id_type=pl.DeviceIdType.MESH,
            )
            r.start()
            rdmas.append(r)
        for r in rdmas:
            r.wait()

        def unpack(ref):
            return ref[0:hd, :], ref[hd:hd + h, :], ref[hd + h:rows, :]

        u1, m1, l1 = unpack(pack_send)
        u2, m2, l2 = unpack(pack_recv.at[0])
        u3, m3, l3 = unpack(pack_recv.at[1])
        u4, m4, l4 = unpack(pack_recv.at[2])
        ma = jnp.maximum(m1, m2)
        mb_ = jnp.maximum(m3, m4)
        a1 = jnp.exp(m1 - ma)
        a2 = jnp.exp(m2 - ma)
        a3 = jnp.exp(m3 - mb_)
        a4 = jnp.exp(m4 - mb_)
        la = l1 * a1 + l2 * a2
        lb_ = l3 * a3 + l4 * a4
        coef = jnp.concatenate([a1, a2, la, a3, a4, lb_], axis=1)
        ce = jnp.dot(onehot, coef, preferred_element_type=jnp.float32)
        o_mine = ((u1 * ce[:, 0:b2] + u2 * ce[:, b2:2 * b2])
                  / ce[:, 2 * b2:3 * b2])
        o_other = ((u3 * ce[:, 3 * b2:4 * b2] + u4 * ce[:, 4 * b2:5 * b2])
                   / ce[:, 5 * b2:6 * b2])
        o_ref[pl.ds(my_y, 1)] = o_mine[None, :, :]
        o_ref[pl.ds(1 - my_y, 1)] = o_other[None, :, :]

    out = pl.pallas_call(
        body,
        out_shape=jax.ShapeDtypeStruct((2, hd, b2), jnp.float32),
        in_specs=[pl.BlockSpec(memory_space=pltpu.VMEM)] * 3,
        out_specs=pl.BlockSpec(memory_space=pltpu.VMEM),
        scratch_shapes=[
            pltpu.VMEM((rows, b2), jnp.float32),
            pltpu.VMEM((3, rows, b2), jnp.float32),
            pltpu.SemaphoreType.DMA((3,)),
            pltpu.SemaphoreType.DMA((3,)),
        ],
        compiler_params=pltpu.CompilerParams(collective_id=0),
    )(Q, KTh, VTh)
    return out.transpose(0, 2, 1).reshape(b, q, h, d)


# device time: 17478 ns/iter; 1.4584x vs baseline; 1.0095x over previous
import jax
import jax.numpy as jnp
from jax import lax
from jax.experimental import pallas as pl
from jax.experimental.pallas import tpu as pltpu


def kernel(Q, K, V):
    b, q, h, d = Q.shape
    _, kv, _, _ = K.shape
    hd = h * d
    b2 = b // 2
    scale = d ** -0.5
    rows = hd + 2 * h

    KT = K.transpose(0, 2, 3, 1).reshape(b, hd, kv)
    VT = V.transpose(0, 2, 3, 1).reshape(b, hd, kv)

    ysel = lax.axis_index("y")
    KTh = lax.dynamic_slice_in_dim(KT, ysel * b2, b2, axis=0).astype(
        jnp.bfloat16)
    VTh = lax.dynamic_slice_in_dim(VT, ysel * b2, b2, axis=0).astype(
        jnp.bfloat16)

    def body(q_ref, k_ref, v_ref, o_ref,
             pack_send, pack_recv, send_sems, recv_sems):
        my_x = lax.axis_index("x")
        my_y = lax.axis_index("y")
        peers = [(1 - my_x, my_y), (my_x, 1 - my_y), (1 - my_x, 1 - my_y)]

        barrier = pltpu.get_barrier_semaphore()
        for p in peers:
            pl.semaphore_signal(barrier, inc=1, device_id=p,
                                device_id_type=pl.DeviceIdType.MESH)

        bigmask = (lax.broadcasted_iota(jnp.int32, (h, hd), 1) // d
                   == lax.broadcasted_iota(jnp.int32, (h, hd), 0)
                   ).astype(jnp.float32)
        onehot = (lax.broadcasted_iota(jnp.int32, (hd, h), 0) // d
                  == lax.broadcasted_iota(jnp.int32, (hd, h), 1)
                  ).astype(jnp.float32)
        onehot_bf = onehot.astype(jnp.bfloat16)
        ones_kv = jnp.ones((kv, 1), dtype=jnp.bfloat16)

        pbs = []
        for i in range(b2):
            bidx = my_y * b2 + i
            Qb = q_ref[bidx, 0]
            Qrep = jnp.concatenate([Qb] * h, axis=1)
            Qblk_t = (Qrep * bigmask).astype(jnp.bfloat16)
            Sb = jnp.dot(Qblk_t, k_ref[i],
                         preferred_element_type=jnp.float32) * scale
            mb = jnp.max(Sb, axis=1, keepdims=True)
            Pb = jnp.exp(Sb - mb)
            lb = jnp.sum(Pb, axis=1, keepdims=True)
            pbs.append(Pb.astype(jnp.bfloat16))
            pack_send[hd:hd + h, i:i + 1] = mb
            pack_send[hd + h:rows, i:i + 1] = lb

        pcat = jnp.concatenate(pbs, axis=1)
        pexp = jnp.dot(onehot_bf, pcat,
                       preferred_element_type=jnp.float32
                       ).astype(jnp.bfloat16)
        vcat = jnp.concatenate([v_ref[i] for i in range(b2)],
                               axis=1)
        blkones = (lax.broadcasted_iota(jnp.int32, (b2 * kv, b2), 0) // kv
                   == lax.broadcasted_iota(jnp.int32, (b2 * kv, b2), 1)
                   ).astype(jnp.bfloat16)
        u_all = jnp.dot(pexp * vcat, blkones,
                        preferred_element_type=jnp.float32)
        pack_send[0:hd, :] = u_all

        pl.semaphore_wait(barrier, 3)

        rdmas = []
        for s, p in enumerate(peers):
            r = pltpu.make_async_remote_copy(
                src_ref=pack_send, dst_ref=pack_recv.at[s],
                send_sem=send_sems.at[s], recv_sem=recv_sems.at[s],
                device_id=p, device_id_type=pl.DeviceIdType.MESH,
            )
            r.start()
            rdmas.append(r)

        def unpack(ref):
            return ref[0:hd, :], ref[hd:hd + h, :], ref[hd + h:rows, :]

        def combine(p1, p2):
            u1, m1, l1 = p1
            u2, m2, l2 = p2
            m = jnp.maximum(m1, m2)
            a1 = jnp.exp(m1 - m)
            a2 = jnp.exp(m2 - m)
            l = l1 * a1 + l2 * a2
            coef = jnp.concatenate([a1, a2, l], axis=1)
            ce = jnp.dot(onehot, coef, preferred_element_type=jnp.float32)
            return ((u1 * ce[:, 0:b2] + u2 * ce[:, b2:2 * b2])
                    / ce[:, 2 * b2:3 * b2])

        rdmas[0].wait_recv()
        o_mine = combine(unpack(pack_send), unpack(pack_recv.at[0]))
        o_ref[pl.ds(my_y, 1)] = o_mine[None, :, :]
        rdmas[1].wait_recv()
        rdmas[2].wait_recv()
        o_other = combine(unpack(pack_recv.at[1]), unpack(pack_recv.at[2]))
        o_ref[pl.ds(1 - my_y, 1)] = o_other[None, :, :]
        for r in rdmas:
            r.wait_send()

    out = pl.pallas_call(
        body,
        out_shape=jax.ShapeDtypeStruct((2, hd, b2), jnp.float32),
        in_specs=[pl.BlockSpec(memory_space=pltpu.VMEM)] * 3,
        out_specs=pl.BlockSpec(memory_space=pltpu.VMEM),
        scratch_shapes=[
            pltpu.VMEM((rows, b2), jnp.float32),
            pltpu.VMEM((3, rows, b2), jnp.float32),
            pltpu.SemaphoreType.DMA((3,)),
            pltpu.SemaphoreType.DMA((3,)),
        ],
        compiler_params=pltpu.CompilerParams(collective_id=0),
    )(Q, KTh, VTh)
    return out.transpose(0, 2, 1).reshape(b, q, h, d)
